# baseline (speedup 1.0000x reference)
import functools

import jax
import jax.numpy as jnp
from jax import lax
from jax.experimental import pallas as pl
from jax.experimental.pallas import tpu as pltpu
from jax.experimental.pallas import tpu_sc as plsc

N_NODES = 100000
EMB = 16
NC = 2
NS = 16
NW = NC * NS
SUB = 128
NSUB = 4
CH = SUB * NSUB
G = 196
EPW = G * CH
E_PAD = NW * EPW
N_PAD = 100352
ROWS_PER_TILE = N_PAD // NS
ZCH = 392
NZ = ROWS_PER_TILE // ZCH


def _scatter_kernel():
    idx_rows_pw = EPW // SUB

    mesh = plsc.VectorSubcoreMesh(
        core_axis_name="c", subcore_axis_name="s",
        num_cores=NC, num_subcores=NS)

    @functools.partial(
        pl.kernel,
        out_type=(jax.ShapeDtypeStruct((N_PAD, EMB), jnp.float32),
                  jax.ShapeDtypeStruct((N_PAD, EMB), jnp.float32)),
        mesh=mesh,
        scratch_types=[
            pltpu.VMEM((4, NSUB, SUB), jnp.int32),
            pltpu.VMEM((4, NSUB, SUB), jnp.int32),
            pltpu.VMEM((4, CH), jnp.float32),
            pltpu.VMEM((2, NSUB, SUB, EMB), jnp.float32),
            pltpu.VMEM((ZCH, EMB), jnp.float32),
            pltpu.VMEM_SHARED((N_PAD, EMB), jnp.float32),
            pltpu.SemaphoreType.DMA((NSUB,)),
            pltpu.SemaphoreType.DMA((2,)),
            pltpu.SemaphoreType.DMA((4,)),
        ],
        compiler_params=pltpu.CompilerParams(use_tc_tiling_on_sc=False),
    )
    def scatter(x_hbm, src_hbm, dst_hbm, w_hbm, p0_hbm, p1_hbm,
                src_v, dst_v, w_v, rows_v, z_v, acc_sh, sg, ss, st):
        c = lax.axis_index("c")
        s = lax.axis_index("s")
        wid = c * NS + s
        idx_row0 = wid * idx_rows_pw
        w_base = wid * EPW

        def stage(g, slot):
            r0 = idx_row0 + g * NSUB
            pltpu.async_copy(src_hbm.at[pl.ds(r0, NSUB)], src_v.at[slot],
                             st.at[slot])
            pltpu.async_copy(dst_hbm.at[pl.ds(r0, NSUB)], dst_v.at[slot],
                             st.at[slot])
            pltpu.async_copy(w_hbm.at[pl.ds(w_base + g * CH, CH)],
                             w_v.at[slot], st.at[slot])

        def stage_wait(g, slot):
            r0 = idx_row0 + g * NSUB
            pltpu.make_async_copy(src_hbm.at[pl.ds(r0, NSUB)],
                                  src_v.at[slot], st.at[slot]).wait()
            pltpu.make_async_copy(dst_hbm.at[pl.ds(r0, NSUB)],
                                  dst_v.at[slot], st.at[slot]).wait()
            pltpu.make_async_copy(w_hbm.at[pl.ds(w_base + g * CH, CH)],
                                  w_v.at[slot], st.at[slot]).wait()

        def drain_scatters(q2, slot):
            for j in range(NSUB):
                pltpu.make_async_copy(rows_v.at[q2, j],
                                      acc_sh.at[dst_v.at[slot, j]],
                                      ss.at[q2]).wait()

        stage(0, 0)
        stage(1, 1)

        def zrow(i, _):
            z_v[i, :] = jnp.zeros((EMB,), jnp.float32)
            return 0
        lax.fori_loop(0, ZCH, zrow, 0)
        row0 = s * ROWS_PER_TILE
        for k in range(NZ):
            pltpu.sync_copy(z_v, acc_sh.at[pl.ds(row0 + k * ZCH, ZCH)])
        plsc.subcore_barrier()

        def section(h, q4):
            g = h * 4 + q4
            q2 = q4 % 2
            nxt = (q4 + 2) % 4

            def do_drain():
                drain_scatters(q2, nxt)
            if q4 < 2:
                pl.when(h >= 1)(do_drain)
            else:
                do_drain()

            def do_stage():
                stage(g + 2, nxt)
            if q4 < 2:
                do_stage()
            else:
                pl.when(h < G // 4 - 1)(do_stage)

            stage_wait(g, q4)

            descs = []
            for j in range(NSUB):
                descs.append(pltpu.async_copy(
                    x_hbm.at[src_v.at[q4, j]], rows_v.at[q2, j], sg.at[j]))

            for j in range(NSUB):
                descs[j].wait()

                def mgrp(i, _):
                    wv = w_v[q4, pl.ds(j * SUB + i * EMB, EMB)]
                    base = i * EMB
                    for k in range(EMB):
                        rows_v[q2, j, base + k, :] = (
                            rows_v[q2, j, base + k, :] * wv[k])
                    return 0
                lax.fori_loop(0, SUB // EMB, mgrp, 0)
                pltpu.async_copy(rows_v.at[q2, j],
                                 acc_sh.at[dst_v.at[q4, j]],
                                 ss.at[q2], add=True)

        def outer(h, _):
            for q4 in range(4):
                section(h, q4)
            return 0
        lax.fori_loop(0, G // 4, outer, 0)

        drain_scatters(0, 2)
        drain_scatters(1, 3)
        plsc.subcore_barrier()

        for k in range(NZ):
            sl = pl.ds(row0 + k * ZCH, ZCH)

            @pl.when(c == 0)
            def _():
                pltpu.sync_copy(acc_sh.at[sl], p0_hbm.at[sl])

            @pl.when(c == 1)
            def _():
                pltpu.sync_copy(acc_sh.at[sl], p1_hbm.at[sl])

    return scatter


def _combine(x, p0, p1):
    rows = (N_NODES * EMB) // 512
    rows_pad = (N_PAD * EMB) // 512

    def body(x_ref, a_ref, b_ref, o_ref):
        o_ref[...] = (x_ref[...] + a_ref[:rows, :] + b_ref[:rows, :]) * 0.5

    spec = pl.BlockSpec((rows, 128), lambda i: (0, i))
    pspec = pl.BlockSpec((rows_pad, 128), lambda i: (0, i))
    out = pl.pallas_call(
        body,
        out_shape=jax.ShapeDtypeStruct((rows, 512), jnp.float32),
        grid=(4,),
        in_specs=[spec, pspec, pspec],
        out_specs=spec,
    )(x.reshape(rows, 512), p0.reshape(rows_pad, 512),
      p1.reshape(rows_pad, 512))
    return out.reshape(N_NODES, EMB)


def kernel(x, edge_index, edge_weight):
    e = edge_weight.shape[0]
    src = edge_index[0]
    dst = edge_index[1]
    pad = E_PAD - e
    if pad > 0:
        spread = (jnp.arange(pad, dtype=jnp.int32) * 37) % N_NODES
        src = jnp.concatenate([src, spread])
        dst = jnp.concatenate([dst, spread])
        edge_weight = jnp.concatenate(
            [edge_weight, jnp.zeros((pad,), jnp.float32)])
    src2d = src.reshape(E_PAD // SUB, SUB)
    dst2d = dst.reshape(E_PAD // SUB, SUB)
    p0, p1 = _scatter_kernel()(x, src2d, dst2d, edge_weight)
    return _combine(x, p0, p1)


# speedup vs baseline: 1.0330x; 1.0330x over previous
import functools

import jax
import jax.numpy as jnp
from jax import lax
from jax.experimental import pallas as pl
from jax.experimental.pallas import tpu as pltpu
from jax.experimental.pallas import tpu_sc as plsc

N_NODES = 100000
EMB = 16
NC = 2
NS = 16
NW = NC * NS
SUB = 128
NSUB = 4
CH = SUB * NSUB
G = 196
EPW = G * CH
E_PAD = NW * EPW
N_PAD = 100352
ROWS_PER_TILE = N_PAD // NS
ZCH = 392
NZ = ROWS_PER_TILE // ZCH


def _scatter_kernel():
    idx_rows_pw = EPW // SUB

    mesh = plsc.VectorSubcoreMesh(
        core_axis_name="c", subcore_axis_name="s",
        num_cores=NC, num_subcores=NS)

    @functools.partial(
        pl.kernel,
        out_type=(jax.ShapeDtypeStruct((N_PAD, EMB), jnp.float32),
                  jax.ShapeDtypeStruct((N_PAD, EMB), jnp.float32)),
        mesh=mesh,
        scratch_types=[
            pltpu.VMEM((4, NSUB, SUB), jnp.int32),
            pltpu.VMEM((4, NSUB, SUB), jnp.int32),
            pltpu.VMEM((4, CH), jnp.float32),
            pltpu.VMEM((2, NSUB, SUB, EMB), jnp.float32),
            pltpu.VMEM((ZCH, EMB), jnp.float32),
            pltpu.VMEM_SHARED((N_PAD, EMB), jnp.float32),
            pltpu.SemaphoreType.DMA((NSUB,)),
            pltpu.SemaphoreType.DMA((2,)),
            pltpu.SemaphoreType.DMA((4,)),
        ],
        compiler_params=pltpu.CompilerParams(use_tc_tiling_on_sc=False),
    )
    def scatter(x_hbm, src_hbm, dst_hbm, w_hbm, p0_hbm, p1_hbm,
                src_v, dst_v, w_v, rows_v, z_v, acc_sh, sg, ss, st):
        c = lax.axis_index("c")
        s = lax.axis_index("s")
        wid = c * NS + s
        idx_row0 = wid * idx_rows_pw
        w_base = wid * EPW

        def stage(g, slot):
            r0 = idx_row0 + g * NSUB
            pltpu.async_copy(src_hbm.at[pl.ds(r0, NSUB)], src_v.at[slot],
                             st.at[slot])
            pltpu.async_copy(dst_hbm.at[pl.ds(r0, NSUB)], dst_v.at[slot],
                             st.at[slot])
            pltpu.async_copy(w_hbm.at[pl.ds(w_base + g * CH, CH)],
                             w_v.at[slot], st.at[slot])

        def stage_wait(g, slot):
            r0 = idx_row0 + g * NSUB
            pltpu.make_async_copy(src_hbm.at[pl.ds(r0, NSUB)],
                                  src_v.at[slot], st.at[slot]).wait()
            pltpu.make_async_copy(dst_hbm.at[pl.ds(r0, NSUB)],
                                  dst_v.at[slot], st.at[slot]).wait()
            pltpu.make_async_copy(w_hbm.at[pl.ds(w_base + g * CH, CH)],
                                  w_v.at[slot], st.at[slot]).wait()

        def drain_scatters(q2, slot):
            for j in range(NSUB):
                pltpu.make_async_copy(rows_v.at[q2, j],
                                      acc_sh.at[dst_v.at[slot, j]],
                                      ss.at[q2]).wait()

        stage(0, 0)
        stage(1, 1)

        def zrow(i, _):
            z_v[i, :] = jnp.zeros((EMB,), jnp.float32)
            return 0
        lax.fori_loop(0, ZCH, zrow, 0)
        row0 = s * ROWS_PER_TILE
        for k in range(NZ):
            pltpu.sync_copy(z_v, acc_sh.at[pl.ds(row0 + k * ZCH, ZCH)])
        plsc.subcore_barrier()

        def section(h, q4):
            g = h * 4 + q4
            q2 = q4 % 2
            nxt = (q4 + 2) % 4

            def do_drain():
                drain_scatters(q2, nxt)
            if q4 < 2:
                pl.when(h >= 1)(do_drain)
            else:
                do_drain()

            def do_stage():
                stage(g + 2, nxt)
            if q4 < 2:
                do_stage()
            else:
                pl.when(h < G // 4 - 1)(do_stage)

            stage_wait(g, q4)

            descs = []
            for j in range(NSUB):
                descs.append(pltpu.async_copy(
                    x_hbm.at[src_v.at[q4, j]], rows_v.at[q2, j], sg.at[j]))

            for j in range(NSUB):
                descs[j].wait()

                def mgrp(i, _):
                    wv = w_v[q4, pl.ds(j * SUB + i * EMB, EMB)]
                    base = i * EMB
                    for k in range(EMB):
                        rows_v[q2, j, base + k, :] = (
                            rows_v[q2, j, base + k, :] * wv[k])
                    return 0
                lax.fori_loop(0, SUB // EMB, mgrp, 0)
                pltpu.async_copy(rows_v.at[q2, j],
                                 acc_sh.at[dst_v.at[q4, j]],
                                 ss.at[q2], add=True)

        def outer(h, _):
            for q4 in range(4):
                section(h, q4)
            return 0
        lax.fori_loop(0, G // 4, outer, 0)

        drain_scatters(0, 2)
        drain_scatters(1, 3)
        plsc.subcore_barrier()

        for k in range(NZ):
            sl = pl.ds(row0 + k * ZCH, ZCH)

            @pl.when(c == 0)
            def _():
                pltpu.sync_copy(acc_sh.at[sl], p0_hbm.at[sl])

            @pl.when(c == 1)
            def _():
                pltpu.sync_copy(acc_sh.at[sl], p1_hbm.at[sl])

    return scatter


def _combine(x, p0, p1):
    rows = (N_NODES * EMB) // 128
    rows_pad = (N_PAD * EMB) // 128

    def body(x_ref, a_ref, b_ref, o_ref):
        o_ref[...] = (x_ref[...] + a_ref[:rows, :] + b_ref[:rows, :]) * 0.5

    out = pl.pallas_call(
        body,
        out_shape=jax.ShapeDtypeStruct((rows, 128), jnp.float32),
    )(x.reshape(rows, 128), p0.reshape(rows_pad, 128),
      p1.reshape(rows_pad, 128))
    return out.reshape(N_NODES, EMB)


def kernel(x, edge_index, edge_weight):
    e = edge_weight.shape[0]
    src = edge_index[0]
    dst = edge_index[1]
    pad = E_PAD - e
    if pad > 0:
        spread = (jnp.arange(pad, dtype=jnp.int32) * 37) % N_NODES
        src = jnp.concatenate([src, spread])
        dst = jnp.concatenate([dst, spread])
        edge_weight = jnp.concatenate(
            [edge_weight, jnp.zeros((pad,), jnp.float32)])
    src2d = src.reshape(E_PAD // SUB, SUB)
    dst2d = dst.reshape(E_PAD // SUB, SUB)
    p0, p1 = _scatter_kernel()(x, src2d, dst2d, edge_weight)
    return _combine(x, p0, p1)
